# baseline (speedup 1.0000x reference)
import functools

import jax
import jax.numpy as jnp
from jax import lax
from jax.experimental import pallas as pl
from jax.experimental.pallas import tpu as pltpu
from jax.experimental.pallas import tpu_sc as plsc

N = 10000
E = 320000
D1 = 128
D2 = 16

NC = 2
NS = 16
NW = NC * NS
EPW = E // NW
CFG = {128: (40, 250, 5), 16: (112, 90, 5)}
NJUNK = 8
RPT = 624
RPT_LAST = N - (NS - 1) * RPT


def _make_sc_agg(D):
    B, STEPS, R = CFG[D]
    GROUPS = STEPS // R
    mesh = plsc.VectorSubcoreMesh(
        core_axis_name="c", subcore_axis_name="s", num_cores=NC, num_subcores=NS
    )

    @functools.partial(
        pl.kernel,
        out_type=jax.ShapeDtypeStruct((NC, N, D), jnp.float32),
        mesh=mesh,
        scratch_types=[
            pltpu.VMEM((STEPS, B), jnp.int32),
            pltpu.VMEM((STEPS, B), jnp.int32),
            [pltpu.VMEM((B, D), jnp.float32) for _ in range(R)],
            pltpu.VMEM_SHARED((N + NJUNK, D), jnp.float32),
            pltpu.SemaphoreType.DMA((R,)),
            pltpu.SemaphoreType.DMA((R,)),
        ],
        compiler_params=pltpu.CompilerParams(use_tc_tiling_on_sc=False),
    )
    def sc_agg(y_hbm, src_hbm, dst_hbm, zero_hbm, out_hbm,
               src_v, dst_v, rows, acc, gsem, ssem):
        c = lax.axis_index("c")
        s = lax.axis_index("s")
        wid = s * NC + c
        row0 = s * RPT

        @pl.when(s < NS - 1)
        def _():
            pltpu.async_copy(zero_hbm.at[pl.ds(row0, RPT)],
                             acc.at[pl.ds(row0, RPT)], ssem.at[0])

        @pl.when(s == NS - 1)
        def _():
            pltpu.async_copy(zero_hbm.at[pl.ds(row0, RPT_LAST)],
                             acc.at[pl.ds(row0, RPT_LAST)], ssem.at[0])

        pltpu.async_copy(src_hbm.at[wid], src_v, gsem.at[0])
        pltpu.async_copy(dst_hbm.at[wid], dst_v, gsem.at[1])
        pltpu.make_async_copy(src_hbm.at[wid], src_v, gsem.at[0]).wait()
        pltpu.make_async_copy(dst_hbm.at[wid], dst_v, gsem.at[1]).wait()

        @pl.when(s < NS - 1)
        def _():
            pltpu.make_async_copy(zero_hbm.at[pl.ds(row0, RPT)],
                                  acc.at[pl.ds(row0, RPT)], ssem.at[0]).wait()

        @pl.when(s == NS - 1)
        def _():
            pltpu.make_async_copy(zero_hbm.at[pl.ds(row0, RPT_LAST)],
                                  acc.at[pl.ds(row0, RPT_LAST)],
                                  ssem.at[0]).wait()

        plsc.subcore_barrier()

        for b in range(R):
            pltpu.async_copy(y_hbm.at[src_v.at[b]], rows[b], gsem.at[b])

        def group(g, carry):
            descs = []
            for b in range(R):
                j = g * R + b
                pltpu.make_async_copy(
                    y_hbm.at[src_v.at[j]], rows[b], gsem.at[b]).wait()
                descs.append(pltpu.async_copy(
                    rows[b], acc.at[dst_v.at[j]], ssem.at[b], add=True))
            for b in range(R):
                descs[b].wait()

                @pl.when(g + 1 < GROUPS)
                def _():
                    jn = (g + 1) * R + b
                    pltpu.async_copy(
                        y_hbm.at[src_v.at[jn]], rows[b], gsem.at[b])
            return carry

        lax.fori_loop(0, GROUPS, group, 0)
        plsc.subcore_barrier()

        @pl.when(s < NS - 1)
        def _():
            pltpu.sync_copy(acc.at[pl.ds(row0, RPT)],
                            out_hbm.at[c, pl.ds(row0, RPT)])

        @pl.when(s == NS - 1)
        def _():
            pltpu.sync_copy(acc.at[pl.ds(row0, RPT_LAST)],
                            out_hbm.at[c, pl.ds(row0, RPT_LAST)])

    return sc_agg


_sc_agg_d1 = _make_sc_agg(D1)
_sc_agg_d2 = _make_sc_agg(D2)


def _mid_body(p_ref, b1_ref, w1_ref, w2_ref, o_ref):
    agg = p_ref[0] + p_ref[1]
    h = jnp.maximum(
        jnp.dot(agg, w1_ref[...], preferred_element_type=jnp.float32)
        + b1_ref[...], 0.0)
    o_ref[...] = jnp.dot(h, w2_ref[...], preferred_element_type=jnp.float32)


def _fin_body(q_ref, b2_ref, o_ref):
    o_ref[...] = q_ref[0] + q_ref[1] + b2_ref[...]


def kernel(x, edge_index, W1, b1, W2, b2):
    ei = edge_index.astype(jnp.int32)
    b_1, s_1, _ = CFG[D1]
    b_2, s_2, _ = CFG[D2]
    src1 = ei[0].reshape(NW, s_1, b_1)
    dst1 = ei[1].reshape(NW, s_1, b_1)
    epad = NW * s_2 * b_2 - E
    src2 = jnp.concatenate(
        [ei[0], jnp.zeros((epad,), jnp.int32)]).reshape(NW, s_2, b_2)
    dst2 = jnp.concatenate(
        [ei[1], jnp.full((epad,), N, jnp.int32)]).reshape(NW, s_2, b_2)
    z1 = jnp.zeros((N, D1), jnp.float32)
    z2 = jnp.zeros((N, D2), jnp.float32)

    p = _sc_agg_d1(x, src1, dst1, z1)
    y2 = pl.pallas_call(
        _mid_body,
        grid=(10,),
        in_specs=[
            pl.BlockSpec((NC, N // 10, D1), lambda i: (0, i, 0)),
            pl.BlockSpec((1, D1), lambda i: (0, 0)),
            pl.BlockSpec((D1, D1), lambda i: (0, 0)),
            pl.BlockSpec((D1, D2), lambda i: (0, 0)),
        ],
        out_specs=pl.BlockSpec((N // 10, D2), lambda i: (i, 0)),
        out_shape=jax.ShapeDtypeStruct((N, D2), jnp.float32),
    )(p, b1.reshape(1, D1), W1, W2)
    q = _sc_agg_d2(y2, src2, dst2, z2)
    qp = q.reshape(NC, N * D2 // 128, 128)
    b2p = jnp.tile(b2, 128 // D2).reshape(1, 128)
    outp = pl.pallas_call(
        _fin_body,
        out_shape=jax.ShapeDtypeStruct((N * D2 // 128, 128), jnp.float32),
    )(qp, b2p)
    return outp.reshape(N, D2)


# speedup vs baseline: 1.0630x; 1.0630x over previous
import functools

import jax
import jax.numpy as jnp
from jax import lax
from jax.experimental import pallas as pl
from jax.experimental.pallas import tpu as pltpu
from jax.experimental.pallas import tpu_sc as plsc

N = 10000
E = 320000
D1 = 128
D2 = 16

NC = 2
NS = 16
NW = NC * NS
EPW = E // NW
CFG = {128: (40, 250, 5), 16: (80, 125, 5)}
NJUNK = 8
RPT = 624
RPT_LAST = N - (NS - 1) * RPT


def _make_sc_agg(D):
    B, STEPS, R = CFG[D]
    GROUPS = STEPS // R
    mesh = plsc.VectorSubcoreMesh(
        core_axis_name="c", subcore_axis_name="s", num_cores=NC, num_subcores=NS
    )

    @functools.partial(
        pl.kernel,
        out_type=jax.ShapeDtypeStruct((NC, N, D), jnp.float32),
        mesh=mesh,
        scratch_types=[
            pltpu.VMEM((STEPS, B), jnp.int32),
            pltpu.VMEM((STEPS, B), jnp.int32),
            [pltpu.VMEM((B, D), jnp.float32) for _ in range(R)],
            pltpu.VMEM_SHARED((N + NJUNK, D), jnp.float32),
            pltpu.SemaphoreType.DMA((R,)),
            pltpu.SemaphoreType.DMA((R,)),
        ],
        compiler_params=pltpu.CompilerParams(use_tc_tiling_on_sc=False),
    )
    def sc_agg(y_hbm, src_hbm, dst_hbm, zero_hbm, out_hbm,
               src_v, dst_v, rows, acc, gsem, ssem):
        c = lax.axis_index("c")
        s = lax.axis_index("s")
        wid = s * NC + c
        row0 = s * RPT

        @pl.when(s < NS - 1)
        def _():
            pltpu.async_copy(zero_hbm.at[pl.ds(row0, RPT)],
                             acc.at[pl.ds(row0, RPT)], ssem.at[0])

        @pl.when(s == NS - 1)
        def _():
            pltpu.async_copy(zero_hbm.at[pl.ds(row0, RPT_LAST)],
                             acc.at[pl.ds(row0, RPT_LAST)], ssem.at[0])

        pltpu.async_copy(src_hbm.at[wid], src_v, gsem.at[0])
        pltpu.async_copy(dst_hbm.at[wid], dst_v, gsem.at[1])
        pltpu.make_async_copy(src_hbm.at[wid], src_v, gsem.at[0]).wait()
        pltpu.make_async_copy(dst_hbm.at[wid], dst_v, gsem.at[1]).wait()

        @pl.when(s < NS - 1)
        def _():
            pltpu.make_async_copy(zero_hbm.at[pl.ds(row0, RPT)],
                                  acc.at[pl.ds(row0, RPT)], ssem.at[0]).wait()

        @pl.when(s == NS - 1)
        def _():
            pltpu.make_async_copy(zero_hbm.at[pl.ds(row0, RPT_LAST)],
                                  acc.at[pl.ds(row0, RPT_LAST)],
                                  ssem.at[0]).wait()

        plsc.subcore_barrier()

        for b in range(R):
            pltpu.async_copy(y_hbm.at[src_v.at[b]], rows[b], gsem.at[b])

        def group(g, carry):
            descs = []
            for b in range(R):
                j = g * R + b
                pltpu.make_async_copy(
                    y_hbm.at[src_v.at[j]], rows[b], gsem.at[b]).wait()
                descs.append(pltpu.async_copy(
                    rows[b], acc.at[dst_v.at[j]], ssem.at[b], add=True))
            for b in range(R):
                descs[b].wait()

                @pl.when(g + 1 < GROUPS)
                def _():
                    jn = (g + 1) * R + b
                    pltpu.async_copy(
                        y_hbm.at[src_v.at[jn]], rows[b], gsem.at[b])
            return carry

        lax.fori_loop(0, GROUPS, group, 0)
        plsc.subcore_barrier()

        @pl.when(s < NS - 1)
        def _():
            pltpu.sync_copy(acc.at[pl.ds(row0, RPT)],
                            out_hbm.at[c, pl.ds(row0, RPT)])

        @pl.when(s == NS - 1)
        def _():
            pltpu.sync_copy(acc.at[pl.ds(row0, RPT_LAST)],
                            out_hbm.at[c, pl.ds(row0, RPT_LAST)])

    return sc_agg


_sc_agg_d1 = _make_sc_agg(D1)
_sc_agg_d2 = _make_sc_agg(D2)


def _mid_body(p_ref, b1_ref, w1_ref, w2_ref, o_ref):
    agg = p_ref[0] + p_ref[1]
    h = jnp.maximum(
        jnp.dot(agg, w1_ref[...], preferred_element_type=jnp.float32)
        + b1_ref[...], 0.0)
    o_ref[...] = jnp.dot(h, w2_ref[...], preferred_element_type=jnp.float32)


def _fin_body(q_ref, b2_ref, o_ref):
    o_ref[...] = q_ref[0] + q_ref[1] + b2_ref[...]


def kernel(x, edge_index, W1, b1, W2, b2):
    ei = edge_index.astype(jnp.int32)
    b_1, s_1, _ = CFG[D1]
    b_2, s_2, _ = CFG[D2]
    src1 = ei[0].reshape(NW, s_1, b_1)
    dst1 = ei[1].reshape(NW, s_1, b_1)
    src2 = ei[0].reshape(NW, s_2, b_2)
    dst2 = ei[1].reshape(NW, s_2, b_2)
    z1 = jnp.zeros((N, D1), jnp.float32)
    z2 = jnp.zeros((N, D2), jnp.float32)

    p = _sc_agg_d1(x, src1, dst1, z1)
    y2 = pl.pallas_call(
        _mid_body,
        out_shape=jax.ShapeDtypeStruct((N, D2), jnp.float32),
    )(p, b1.reshape(1, D1), W1, W2)
    q = _sc_agg_d2(y2, src2, dst2, z2)
    qp = q.reshape(NC, N * D2 // 128, 128)
    b2p = jnp.tile(b2, 128 // D2).reshape(1, 128)
    outp = pl.pallas_call(
        _fin_body,
        out_shape=jax.ShapeDtypeStruct((N * D2 // 128, 128), jnp.float32),
    )(qp, b2p)
    return outp.reshape(N, D2)
